# baseline (speedup 1.0000x reference)
import functools

import jax
import jax.numpy as jnp
from jax import lax
from jax.experimental import pallas as pl
from jax.experimental.pallas import tpu as pltpu
from jax.experimental.pallas import tpu_sc as plsc

_D, _N, _K = 128, 10000, 32
_NC, _NS, _L = 2, 16, 16
_NW = _NC * _NS
_DPW = _D // _NW
_CHUNK = 400
_NCH = _N // _CHUNK
_G = _CHUNK // _L


def _body(f_hbm, nb_hbm, y_hbm, table_v, idx_v, out_v, sems):
    wid = lax.axis_index("s") * _NC + lax.axis_index("c")
    d0 = wid * _DPW

    def idx_copy(c, buf):
        return pltpu.make_async_copy(nb_hbm.at[c], idx_v.at[buf], sems.at[buf])

    idx_copy(0, 0).start()
    pltpu.sync_copy(f_hbm.at[pl.ds(d0 * _N, _DPW * _N)], table_v)

    def chunk_body(c, carry):
        buf = lax.rem(c, 2)

        @pl.when(c + 1 < _NCH)
        def _():
            idx_copy(c + 1, 1 - buf).start()

        idx_copy(c, buf).wait()
        c0 = c * _CHUNK

        @plsc.parallel_loop(0, _G, unroll=2)
        def group_body(g):
            base = g * _L

            def k_block(kb, accs):
                out = []
                for kk in range(16):
                    idx = idx_v[buf, kb * 16 + kk, pl.ds(base, _L)]
                    for dd in range(_DPW):
                        v = plsc.load_gather(table_v, [idx + (dd * _N)])
                        if kk == 0:
                            out.append(jnp.maximum(accs[dd], v))
                        else:
                            out[dd] = jnp.maximum(out[dd], v)
                return tuple(out)

            neg = jnp.full((_L,), -jnp.inf, jnp.float32)
            accs = lax.fori_loop(
                0, _K // 16, k_block, (neg, neg, neg, neg))
            for dd in range(_DPW):
                out_v[dd, pl.ds(c0 + base, _L)] = accs[dd]

        return carry

    lax.fori_loop(0, _NCH, chunk_body, 0)
    pltpu.sync_copy(out_v, y_hbm.at[pl.ds(d0, _DPW)])


_flexpool = functools.partial(
    pl.kernel,
    mesh=plsc.VectorSubcoreMesh(core_axis_name="c", subcore_axis_name="s"),
    out_type=jax.ShapeDtypeStruct((_D, _N), jnp.float32),
    compiler_params=pltpu.CompilerParams(
        needs_layout_passes=False, skip_device_barrier=True),
    scratch_types=[
        pltpu.VMEM((_DPW * _N,), jnp.float32),
        pltpu.VMEM((2, _K, _CHUNK), jnp.int32),
        pltpu.VMEM((_DPW, _N), jnp.float32),
        pltpu.SemaphoreType.DMA((2,)),
    ],
)(_body)


@jax.jit
def kernel(features, neighborhoods):
    f = features.reshape(_D * _N)
    nb = neighborhoods[0].reshape(_K, _NCH, _CHUNK).transpose(1, 0, 2)
    y = _flexpool(f, nb)
    return y[None]


# speedup vs baseline: 1.0469x; 1.0469x over previous
import functools

import jax
import jax.numpy as jnp
from jax import lax
from jax.experimental import pallas as pl
from jax.experimental.pallas import tpu as pltpu
from jax.experimental.pallas import tpu_sc as plsc

_D, _N, _K = 128, 10000, 32
_NC, _NS, _L = 2, 16, 16
_NW = _NC * _NS
_DPW = _D // _NW
_CHUNK = 400
_NCH = _N // _CHUNK
_G = _CHUNK // _L


def _body(f_hbm, nb_hbm, y_hbm, table_v, idx_v, out_v, sems):
    wid = lax.axis_index("s") * _NC + lax.axis_index("c")
    d0 = wid * _DPW

    def idx_copy(c, buf):
        return pltpu.make_async_copy(nb_hbm.at[c], idx_v.at[buf], sems.at[buf])

    idx_copy(0, 0).start()
    pltpu.sync_copy(f_hbm.at[pl.ds(d0 * _N, _DPW * _N)], table_v)

    def chunk_body(c, carry):
        buf = lax.rem(c, 2)

        @pl.when(c + 1 < _NCH)
        def _():
            idx_copy(c + 1, 1 - buf).start()

        idx_copy(c, buf).wait()
        c0 = c * _CHUNK

        @plsc.parallel_loop(0, _G, unroll=4)
        def group_body(g):
            base = g * _L

            def k_block(kb, accs):
                out = []
                for kk in range(8):
                    idx = idx_v[buf, kb * 8 + kk, pl.ds(base, _L)]
                    for dd in range(_DPW):
                        v = plsc.load_gather(table_v, [idx + (dd * _N)])
                        if kk == 0:
                            out.append(jnp.maximum(accs[dd], v))
                        else:
                            out[dd] = jnp.maximum(out[dd], v)
                return tuple(out)

            neg = jnp.full((_L,), -jnp.inf, jnp.float32)
            accs = lax.fori_loop(
                0, _K // 8, k_block, (neg, neg, neg, neg))
            for dd in range(_DPW):
                out_v[dd, pl.ds(c0 + base, _L)] = accs[dd]

        return carry

    lax.fori_loop(0, _NCH, chunk_body, 0)
    pltpu.sync_copy(out_v, y_hbm.at[pl.ds(d0, _DPW)])


_flexpool = functools.partial(
    pl.kernel,
    mesh=plsc.VectorSubcoreMesh(core_axis_name="c", subcore_axis_name="s"),
    out_type=jax.ShapeDtypeStruct((_D, _N), jnp.float32),
    compiler_params=pltpu.CompilerParams(
        needs_layout_passes=False, skip_device_barrier=True),
    scratch_types=[
        pltpu.VMEM((_DPW * _N,), jnp.float32),
        pltpu.VMEM((2, _K, _CHUNK), jnp.int32),
        pltpu.VMEM((_DPW, _N), jnp.float32),
        pltpu.SemaphoreType.DMA((2,)),
    ],
)(_body)


@jax.jit
def kernel(features, neighborhoods):
    f = features.reshape(_D * _N)
    nb = neighborhoods[0].reshape(_K, _NCH, _CHUNK).transpose(1, 0, 2)
    y = _flexpool(f, nb)
    return y[None]


# speedup vs baseline: 1.0513x; 1.0042x over previous
import functools

import jax
import jax.numpy as jnp
from jax import lax
from jax.experimental import pallas as pl
from jax.experimental.pallas import tpu as pltpu
from jax.experimental.pallas import tpu_sc as plsc

_D, _N, _K = 128, 10000, 32
_NC, _NS, _L = 2, 16, 16
_NW = _NC * _NS
_DPW = _D // _NW
_CHUNK = 400
_NCH = _N // _CHUNK
_G = _CHUNK // _L


def _body(f_hbm, nb_hbm, y_hbm, table_v, idx_v, out_v, sems):
    wid = lax.axis_index("s") * _NC + lax.axis_index("c")
    d0 = wid * _DPW

    def idx_copy(c, buf):
        return pltpu.make_async_copy(nb_hbm.at[c], idx_v.at[buf], sems.at[buf])

    idx_copy(0, 0).start()
    pltpu.sync_copy(f_hbm.at[pl.ds(d0 * _N, _DPW * _N)], table_v)

    def chunk_body(c, carry):
        buf = lax.rem(c, 2)

        @pl.when(c + 1 < _NCH)
        def _():
            idx_copy(c + 1, 1 - buf).start()

        idx_copy(c, buf).wait()
        c0 = c * _CHUNK

        @plsc.parallel_loop(0, _G, unroll=5)
        def group_body(g):
            base = g * _L

            def k_block(kb, accs):
                out = []
                for kk in range(8):
                    idx = idx_v[buf, kb * 8 + kk, pl.ds(base, _L)]
                    for dd in range(_DPW):
                        v = plsc.load_gather(table_v, [idx + (dd * _N)])
                        if kk == 0:
                            out.append(jnp.maximum(accs[dd], v))
                        else:
                            out[dd] = jnp.maximum(out[dd], v)
                return tuple(out)

            neg = jnp.full((_L,), -jnp.inf, jnp.float32)
            accs = lax.fori_loop(
                0, _K // 8, k_block, (neg, neg, neg, neg))
            for dd in range(_DPW):
                out_v[dd, pl.ds(c0 + base, _L)] = accs[dd]

        return carry

    lax.fori_loop(0, _NCH, chunk_body, 0)
    pltpu.sync_copy(out_v, y_hbm.at[pl.ds(d0, _DPW)])


_flexpool = functools.partial(
    pl.kernel,
    mesh=plsc.VectorSubcoreMesh(core_axis_name="c", subcore_axis_name="s"),
    out_type=jax.ShapeDtypeStruct((_D, _N), jnp.float32),
    compiler_params=pltpu.CompilerParams(
        needs_layout_passes=False, skip_device_barrier=True),
    scratch_types=[
        pltpu.VMEM((_DPW * _N,), jnp.float32),
        pltpu.VMEM((2, _K, _CHUNK), jnp.int32),
        pltpu.VMEM((_DPW, _N), jnp.float32),
        pltpu.SemaphoreType.DMA((2,)),
    ],
)(_body)


@jax.jit
def kernel(features, neighborhoods):
    f = features.reshape(_D * _N)
    nb = neighborhoods[0].reshape(_K, _NCH, _CHUNK).transpose(1, 0, 2)
    y = _flexpool(f, nb)
    return y[None]
